# baseline (speedup 1.0000x reference)
import functools

import jax
import jax.numpy as jnp
from jax import lax
from jax.experimental import pallas as pl
from jax.experimental.pallas import tpu as pltpu
from jax.experimental.pallas import tpu_sc as plsc

N_NODES = 10000
D = 128
E = 320000
NC = 2
NS = 16
NW = NC * NS
CHUNK = 128
CHUNKS_PER_TILE = 80
G = 8
NGROUPS = CHUNKS_PER_TILE // G
E_PAD = NW * CHUNKS_PER_TILE * CHUNK
ROWS_PER_TILE = 624
TAIL_ROWS = N_NODES - NS * ROWS_PER_TILE
LANES = 16

_mesh = plsc.VectorSubcoreMesh(core_axis_name="c", subcore_axis_name="s")


@functools.partial(
    pl.kernel,
    out_type=jax.ShapeDtypeStruct((NC, N_NODES, D), jnp.float32),
    mesh=_mesh,
    scratch_types=[
        pltpu.VMEM((G, CHUNK), jnp.int32),
        pltpu.VMEM((G, CHUNK), jnp.int32),
        pltpu.VMEM((G, CHUNK), jnp.float32),
        pltpu.VMEM((2, CHUNK, D), jnp.float32),
        pltpu.VMEM_SHARED((N_NODES, D), jnp.float32),
        pltpu.SemaphoreType.DMA,
        pltpu.SemaphoreType.DMA,
        pltpu.SemaphoreType.DMA,
        pltpu.SemaphoreType.DMA,
    ],
)
def _prop(feats_hbm, src_hbm, dst_hbm, w_hbm, zeros_hbm, out_hbm,
          src_v, dst_v, w_v, rows_v, acc_sh, gsem0, gsem1, ssem0, ssem1):
    cid = lax.axis_index("c")
    sid = lax.axis_index("s")
    wid = sid * NC + cid

    r0 = sid * ROWS_PER_TILE
    pltpu.sync_copy(zeros_hbm.at[pl.ds(r0, ROWS_PER_TILE)],
                    acc_sh.at[pl.ds(r0, ROWS_PER_TILE)])

    @pl.when(sid == 0)
    def _stage_tail():
        t0 = NS * ROWS_PER_TILE
        pltpu.sync_copy(zeros_hbm.at[pl.ds(t0, TAIL_ROWS)],
                        acc_sh.at[pl.ds(t0, TAIL_ROWS)])

    plsc.subcore_barrier()

    def scale(b, j):
        def group_body(g, c2):
            wvec = w_v[j, pl.ds(g * LANES, LANES)]
            for l in range(LANES):
                w_s = wvec[l]
                i = g * LANES + l
                for d in range(D // LANES):
                    sl = pl.ds(d * LANES, LANES)
                    rows_v[b, i, sl] = rows_v[b, i, sl] * w_s
            return c2
        lax.fori_loop(0, CHUNK // LANES, group_body, 0)

    gsems = (gsem0, gsem1)
    ssems = (ssem0, ssem1)

    def group_loop(g, carry):
        pltpu.sync_copy(src_hbm.at[wid, pl.ds(g * G, G)], src_v)
        pltpu.sync_copy(dst_hbm.at[wid, pl.ds(g * G, G)], dst_v)
        pltpu.sync_copy(w_hbm.at[wid, pl.ds(g * G, G)], w_v)

        gh = [None, None]
        sh = [None, None]
        gh[0] = pltpu.async_copy(feats_hbm.at[src_v.at[0]], rows_v.at[0],
                                 gsems[0])
        for c in range(G):
            b = c % 2
            gh[b].wait()
            if c + 1 < G:
                nb = 1 - b
                if sh[nb] is not None:
                    sh[nb].wait()
                gh[nb] = pltpu.async_copy(feats_hbm.at[src_v.at[c + 1]],
                                          rows_v.at[nb], gsems[nb])
            scale(b, c)
            sh[b] = pltpu.async_copy(rows_v.at[b], acc_sh.at[dst_v.at[c]],
                                     ssems[b], add=True)
        sh[0].wait()
        sh[1].wait()
        return carry
    lax.fori_loop(0, NGROUPS, group_loop, 0)

    plsc.subcore_barrier()
    pltpu.sync_copy(acc_sh.at[pl.ds(r0, ROWS_PER_TILE)],
                    out_hbm.at[cid, pl.ds(r0, ROWS_PER_TILE)])

    @pl.when(sid == 0)
    def _write_tail():
        t0 = NS * ROWS_PER_TILE
        pltpu.sync_copy(acc_sh.at[pl.ds(t0, TAIL_ROWS)],
                        out_hbm.at[cid, pl.ds(t0, TAIL_ROWS)])


_BN = 1000


def _combine_h_body(p_ref, o_ref):
    o_ref[...] = p_ref[0] + p_ref[1]


def _combine_h(p):
    return pl.pallas_call(
        _combine_h_body,
        out_shape=jax.ShapeDtypeStruct((N_NODES, D), jnp.float32),
        grid=(N_NODES // _BN,),
        in_specs=[pl.BlockSpec((NC, _BN, D), lambda i: (0, i, 0))],
        out_specs=pl.BlockSpec((_BN, D), lambda i: (i, 0)),
    )(p)


def _final_body(x_ref, h1_ref, q_ref, o_ref):
    o_ref[...] = (x_ref[...] + h1_ref[...] + q_ref[0] + q_ref[1]) * (1.0 / 3.0)


def _final(x, h1, q):
    return pl.pallas_call(
        _final_body,
        out_shape=jax.ShapeDtypeStruct((N_NODES, D), jnp.float32),
        grid=(N_NODES // _BN,),
        in_specs=[
            pl.BlockSpec((_BN, D), lambda i: (i, 0)),
            pl.BlockSpec((_BN, D), lambda i: (i, 0)),
            pl.BlockSpec((NC, _BN, D), lambda i: (0, i, 0)),
        ],
        out_specs=pl.BlockSpec((_BN, D), lambda i: (i, 0)),
    )(x, h1, q)


def kernel(x, edge_weight, edge_index):
    pad = E_PAD - E
    src = jnp.pad(edge_index[0].astype(jnp.int32), (0, pad))
    dst = jnp.pad(edge_index[1].astype(jnp.int32), (0, pad))
    w = jnp.pad(edge_weight.astype(jnp.float32), (0, pad))
    src = src.reshape(NW, CHUNKS_PER_TILE, CHUNK)
    dst = dst.reshape(NW, CHUNKS_PER_TILE, CHUNK)
    w = w.reshape(NW, CHUNKS_PER_TILE, CHUNK)
    zeros = jnp.zeros((N_NODES, D), jnp.float32)

    xf = x.astype(jnp.float32)
    p = _prop(xf, src, dst, w, zeros)
    h1 = _combine_h(p)
    q = _prop(h1, src, dst, w, zeros)
    return _final(xf, h1, q)


# speedup vs baseline: 1.1646x; 1.1646x over previous
import functools

import jax
import jax.numpy as jnp
from jax import lax
from jax.experimental import pallas as pl
from jax.experimental.pallas import tpu as pltpu
from jax.experimental.pallas import tpu_sc as plsc

N_NODES = 10000
D = 128
E = 320000
NC = 2
NS = 16
CHUNK = 128
G = 8
C0 = 120
C1 = 40
TOT_CHUNKS = NS * (C0 + C1)
E_PAD = TOT_CHUNKS * CHUNK
ROWS_PER_TILE = 624
TAIL_ROWS = N_NODES - NS * ROWS_PER_TILE
LANES = 16

_mesh = plsc.VectorSubcoreMesh(core_axis_name="c", subcore_axis_name="s")


@functools.partial(
    pl.kernel,
    out_type=jax.ShapeDtypeStruct((NC, N_NODES, D), jnp.float32),
    mesh=_mesh,
    scratch_types=[
        pltpu.VMEM((G, CHUNK), jnp.int32),
        pltpu.VMEM((G, CHUNK), jnp.int32),
        pltpu.VMEM((G, CHUNK), jnp.float32),
        pltpu.VMEM((2, CHUNK, D), jnp.float32),
        pltpu.VMEM_SHARED((N_NODES, D), jnp.float32),
        pltpu.SemaphoreType.DMA,
        pltpu.SemaphoreType.DMA,
        pltpu.SemaphoreType.DMA,
        pltpu.SemaphoreType.DMA,
    ],
)
def _prop(feats_hbm, src_hbm, dst_hbm, w_hbm, zeros_hbm, out_hbm,
          src_v, dst_v, w_v, rows_v, acc_sh, gsem0, gsem1, ssem0, ssem1):
    cid = lax.axis_index("c")
    sid = lax.axis_index("s")
    base = jnp.where(cid == 0, sid * C0, NS * C0 + sid * C1)
    ngroups = jnp.where(cid == 0, C0 // G, C1 // G)

    r0 = sid * ROWS_PER_TILE
    pltpu.sync_copy(zeros_hbm.at[pl.ds(r0, ROWS_PER_TILE)],
                    acc_sh.at[pl.ds(r0, ROWS_PER_TILE)])

    @pl.when(sid == 0)
    def _stage_tail():
        t0 = NS * ROWS_PER_TILE
        pltpu.sync_copy(zeros_hbm.at[pl.ds(t0, TAIL_ROWS)],
                        acc_sh.at[pl.ds(t0, TAIL_ROWS)])

    plsc.subcore_barrier()

    def scale(b, j):
        def group_body(g, c2):
            wvec = w_v[j, pl.ds(g * LANES, LANES)]
            for l in range(LANES):
                w_s = wvec[l]
                i = g * LANES + l
                for d in range(D // LANES):
                    sl = pl.ds(d * LANES, LANES)
                    rows_v[b, i, sl] = rows_v[b, i, sl] * w_s
            return c2
        lax.fori_loop(0, CHUNK // LANES, group_body, 0)

    gsems = (gsem0, gsem1)
    ssems = (ssem0, ssem1)

    def group_loop(g, carry):
        g0 = base + g * G
        pltpu.sync_copy(src_hbm.at[pl.ds(g0, G)], src_v)
        pltpu.sync_copy(dst_hbm.at[pl.ds(g0, G)], dst_v)
        pltpu.sync_copy(w_hbm.at[pl.ds(g0, G)], w_v)

        gh = [None, None]
        sh = [None, None]
        gh[0] = pltpu.async_copy(feats_hbm.at[src_v.at[0]], rows_v.at[0],
                                 gsems[0])
        for c in range(G):
            b = c % 2
            gh[b].wait()
            if c + 1 < G:
                nb = 1 - b
                if sh[nb] is not None:
                    sh[nb].wait()
                gh[nb] = pltpu.async_copy(feats_hbm.at[src_v.at[c + 1]],
                                          rows_v.at[nb], gsems[nb])
            scale(b, c)
            sh[b] = pltpu.async_copy(rows_v.at[b], acc_sh.at[dst_v.at[c]],
                                     ssems[b], add=True)
        sh[0].wait()
        sh[1].wait()
        return carry
    lax.fori_loop(0, ngroups, group_loop, 0)

    plsc.subcore_barrier()
    pltpu.sync_copy(acc_sh.at[pl.ds(r0, ROWS_PER_TILE)],
                    out_hbm.at[cid, pl.ds(r0, ROWS_PER_TILE)])

    @pl.when(sid == 0)
    def _write_tail():
        t0 = NS * ROWS_PER_TILE
        pltpu.sync_copy(acc_sh.at[pl.ds(t0, TAIL_ROWS)],
                        out_hbm.at[cid, pl.ds(t0, TAIL_ROWS)])


_BN = 1000


def _combine_h_body(p_ref, o_ref):
    o_ref[...] = p_ref[0] + p_ref[1]


def _combine_h(p):
    return pl.pallas_call(
        _combine_h_body,
        out_shape=jax.ShapeDtypeStruct((N_NODES, D), jnp.float32),
        grid=(N_NODES // _BN,),
        in_specs=[pl.BlockSpec((NC, _BN, D), lambda i: (0, i, 0))],
        out_specs=pl.BlockSpec((_BN, D), lambda i: (i, 0)),
    )(p)


def _final_body(x_ref, h1_ref, q_ref, o_ref):
    o_ref[...] = (x_ref[...] + h1_ref[...] + q_ref[0] + q_ref[1]) * (1.0 / 3.0)


def _final(x, h1, q):
    return pl.pallas_call(
        _final_body,
        out_shape=jax.ShapeDtypeStruct((N_NODES, D), jnp.float32),
        grid=(N_NODES // _BN,),
        in_specs=[
            pl.BlockSpec((_BN, D), lambda i: (i, 0)),
            pl.BlockSpec((_BN, D), lambda i: (i, 0)),
            pl.BlockSpec((NC, _BN, D), lambda i: (0, i, 0)),
        ],
        out_specs=pl.BlockSpec((_BN, D), lambda i: (i, 0)),
    )(x, h1, q)


def kernel(x, edge_weight, edge_index):
    pad = E_PAD - E
    src = jnp.pad(edge_index[0].astype(jnp.int32), (0, pad))
    dst = jnp.pad(edge_index[1].astype(jnp.int32), (0, pad))
    w = jnp.pad(edge_weight.astype(jnp.float32), (0, pad))
    src = src.reshape(TOT_CHUNKS, CHUNK)
    dst = dst.reshape(TOT_CHUNKS, CHUNK)
    w = w.reshape(TOT_CHUNKS, CHUNK)
    zeros = jnp.zeros((N_NODES, D), jnp.float32)

    xf = x.astype(jnp.float32)
    p = _prop(xf, src, dst, w, zeros)
    h1 = _combine_h(p)
    q = _prop(h1, src, dst, w, zeros)
    return _final(xf, h1, q)
